# Initial kernel scaffold; baseline (speedup 1.0000x reference)
import jax
import jax.numpy as jnp
from jax.experimental import pallas as pl


def kernel(x, edge_index, W1_l, b1_l, W1_r, W2_l, b2_l, W2_r, W_dec, b_dec):
    raise NotImplementedError("write your pallas kernel here")


# speedup vs baseline: 6.9080x; 6.9080x over previous
import functools

import jax
import jax.numpy as jnp
from jax import lax
from jax.experimental import pallas as pl
from jax.experimental.pallas import tpu as pltpu
from jax.experimental.pallas import tpu_sc as plsc

N = 10000
NPAD = 10240
E = 320000
NC = 2
NS = 16
RPT = NPAD // NS
EPW = E // (NC * NS)
B = 80
NCHUNK = EPW // B

_f32 = jnp.float32


def _make_seg_sum(D, with_deg):
  mesh = plsc.VectorSubcoreMesh(core_axis_name="c", subcore_axis_name="s")
  out_type = [jax.ShapeDtypeStruct((NC, NPAD, D), _f32)]
  scratch = [
      pltpu.VMEM((1, B), jnp.int32),
      pltpu.VMEM((1, B), jnp.int32),
      pltpu.VMEM((B, D), _f32),
      pltpu.VMEM((RPT, D), _f32),
      pltpu.VMEM_SHARED((NPAD, D), _f32),
      pltpu.SemaphoreType.DMA,
  ]
  if with_deg:
    out_type.append(jax.ShapeDtypeStruct((NC, NPAD), _f32))
    scratch += [
        pltpu.VMEM((B,), _f32),
        pltpu.VMEM((RPT,), _f32),
        pltpu.VMEM_SHARED((NPAD,), _f32),
    ]

  def body(p_hbm, src_hbm, dst_hbm, *rest):
    if with_deg:
      (acc_out, deg_out, src_v, dst_v, rows_v, zbuf, acc_sh, sem,
       ones_v, zdeg, deg_sh) = rest
    else:
      acc_out, src_v, dst_v, rows_v, zbuf, acc_sh, sem = rest
    c = lax.axis_index("c")
    s = lax.axis_index("s")
    wid = c * NS + s

    zeros16 = jnp.zeros((16,), _f32)
    ones16 = jnp.ones((16,), _f32)

    def zrow(r, carry):
      for k in range(D // 16):
        zbuf[r, pl.ds(k * 16, 16)] = zeros16
      return carry

    lax.fori_loop(0, RPT, zrow, 0)
    row0 = pl.multiple_of(s * RPT, 8)
    pltpu.sync_copy(zbuf, acc_sh.at[pl.ds(row0, RPT)])
    if with_deg:
      def zdrow(r, carry):
        zdeg[pl.ds(pl.multiple_of(r * 16, 16), 16)] = zeros16
        return carry
      lax.fori_loop(0, RPT // 16, zdrow, 0)
      pltpu.sync_copy(zdeg, deg_sh.at[pl.ds(row0, RPT)])
      for k in range(B // 16):
        ones_v[pl.ds(k * 16, 16)] = ones16
    plsc.subcore_barrier()

    def step(j, carry):
      off = pl.multiple_of(wid * EPW + j * B, 8)
      pltpu.sync_copy(src_hbm.at[pl.ds(off, B)], src_v.at[0])
      pltpu.sync_copy(dst_hbm.at[pl.ds(off, B)], dst_v.at[0])
      pltpu.async_copy(p_hbm.at[src_v.at[0]], rows_v, sem).wait()
      pltpu.sync_copy(rows_v, acc_sh.at[dst_v.at[0]], add=True)
      if with_deg:
        pltpu.sync_copy(ones_v, deg_sh.at[dst_v.at[0]], add=True)
      return carry

    lax.fori_loop(0, NCHUNK, step, 0)
    plsc.subcore_barrier()

    pltpu.sync_copy(acc_sh.at[pl.ds(row0, RPT)],
                    acc_out.at[c, pl.ds(row0, RPT)])
    if with_deg:
      pltpu.sync_copy(deg_sh.at[pl.ds(row0, RPT)],
                      deg_out.at[c, pl.ds(row0, RPT)])

  return pl.kernel(body, out_type=out_type, mesh=mesh, scratch_types=scratch,
                   compiler_params=pltpu.CompilerParams(
                       use_tc_tiling_on_sc=False))


_seg_sum_l1 = _make_seg_sum(32, with_deg=True)
_seg_sum_l2 = _make_seg_sum(16, with_deg=False)


def _proj_body(x_ref, wl_ref, wr_ref, l_o, r_o):
  xv = x_ref[...]
  l_o[...] = jnp.dot(xv, wl_ref[...], preferred_element_type=_f32)
  r_o[...] = jnp.dot(xv, wr_ref[...], preferred_element_type=_f32)


def _mid_body(s1p_ref, d0_ref, d1_ref, p1r_ref, b1_ref, w2l_ref, w2r_ref,
              p2l_o, p2r_o, rdeg_o):
  deg = jnp.maximum(d0_ref[...] + d1_ref[...], 1.0)
  rdeg = 1.0 / deg
  s1 = s1p_ref[0, :N, :] + s1p_ref[1, :N, :]
  h = s1 * rdeg + b1_ref[...] + p1r_ref[...]
  h = jnp.maximum(h, 0.0)
  p2l_o[...] = jnp.dot(h, w2l_ref[...], preferred_element_type=_f32)
  p2r_o[...] = jnp.dot(h, w2r_ref[...], preferred_element_type=_f32)
  rdeg_o[...] = rdeg


def _dec_body(s2p_ref, rdeg_ref, p2r_ref, b2_ref, wd_ref, bd_ref,
              xr_o, z_o):
  s2 = s2p_ref[0, :N, :] + s2p_ref[1, :N, :]
  z = s2 * rdeg_ref[...] + b2_ref[...] + p2r_ref[...]
  z_o[...] = z
  xr_o[...] = jnp.dot(z, wd_ref[...], preferred_element_type=_f32) + bd_ref[...]


def kernel(x, edge_index, W1_l, b1_l, W1_r, W2_l, b2_l, W2_r, W_dec, b_dec):
  src = edge_index[0].astype(jnp.int32)
  dst = edge_index[1].astype(jnp.int32)

  p1l, p1r = pl.pallas_call(
      _proj_body,
      out_shape=[jax.ShapeDtypeStruct((N, 32), _f32),
                 jax.ShapeDtypeStruct((N, 32), _f32)],
  )(x, W1_l.T, W1_r.T)

  s1p, degp = _seg_sum_l1(p1l, src, dst)
  d0 = degp[0, :N, None]
  d1 = degp[1, :N, None]

  p2l, p2r, rdeg = pl.pallas_call(
      _mid_body,
      out_shape=[jax.ShapeDtypeStruct((N, 16), _f32),
                 jax.ShapeDtypeStruct((N, 16), _f32),
                 jax.ShapeDtypeStruct((N, 1), _f32)],
  )(s1p, d0, d1, p1r, b1_l[None, :], W2_l.T, W2_r.T)

  (s2p,) = _seg_sum_l2(p2l, src, dst)

  x_recon, z = pl.pallas_call(
      _dec_body,
      out_shape=[jax.ShapeDtypeStruct((N, 128), _f32),
                 jax.ShapeDtypeStruct((N, 16), _f32)],
  )(s2p, rdeg, p2r, b2_l[None, :], W_dec.T, b_dec[None, :])

  return (x_recon, z)


# speedup vs baseline: 20.7089x; 2.9978x over previous
import functools

import jax
import jax.numpy as jnp
from jax import lax
from jax.experimental import pallas as pl
from jax.experimental.pallas import tpu as pltpu
from jax.experimental.pallas import tpu_sc as plsc

N = 10000
NPAD = 10240
E = 320000
NC = 2
NS = 16
RPT = NPAD // NS
NW = NC * NS
EPW = E // NW
B = 100
NCHUNK = EPW // B
SPT = N // NS

_f32 = jnp.float32


def _make_seg_sum(D, with_deg):
  mesh = plsc.VectorSubcoreMesh(core_axis_name="c", subcore_axis_name="s")
  out_type = [jax.ShapeDtypeStruct((NC, NPAD, D), _f32)]
  scratch = [
      pltpu.VMEM((NCHUNK, B), jnp.int32),
      pltpu.VMEM((NCHUNK, B), jnp.int32),
      pltpu.VMEM((B, D), _f32),
      pltpu.VMEM((B, D), _f32),
      pltpu.VMEM((RPT, D), _f32),
      pltpu.VMEM_SHARED((N, D), _f32),
      pltpu.VMEM_SHARED((NPAD, D), _f32),
      pltpu.SemaphoreType.DMA,
      pltpu.SemaphoreType.DMA,
  ]
  if with_deg:
    out_type.append(jax.ShapeDtypeStruct((NC, NPAD), _f32))
    scratch += [
        pltpu.VMEM((112,), _f32),
        pltpu.VMEM((RPT,), _f32),
        pltpu.VMEM_SHARED((NPAD,), _f32),
    ]

  def body(p_hbm, src_hbm, dst_hbm, *rest):
    if with_deg:
      (acc_out, deg_out, src_v, dst_v, rows0, rows1, zbuf, p_sh, acc_sh,
       sem0, sem1, ones_v, zdeg, deg_sh) = rest
    else:
      (acc_out, src_v, dst_v, rows0, rows1, zbuf, p_sh, acc_sh,
       sem0, sem1) = rest
    c = lax.axis_index("c")
    s = lax.axis_index("s")
    wid = c * NS + s

    zeros16 = jnp.zeros((16,), _f32)
    ones16 = jnp.ones((16,), _f32)

    prow = s * SPT
    stage = pltpu.async_copy(p_hbm.at[pl.ds(prow, SPT)],
                             p_sh.at[pl.ds(prow, SPT)], sem0)
    ldsrc = pltpu.async_copy(src_hbm.at[wid], src_v, sem1)

    def zrow(r, carry):
      for k in range(D // 16):
        zbuf[r, pl.ds(k * 16, 16)] = zeros16
      return carry

    lax.fori_loop(0, RPT, zrow, 0)
    row0 = pl.multiple_of(s * RPT, 8)
    pltpu.sync_copy(zbuf, acc_sh.at[pl.ds(row0, RPT)])
    if with_deg:
      def zdrow(r, carry):
        zdeg[pl.ds(pl.multiple_of(r * 16, 16), 16)] = zeros16
        return carry
      lax.fori_loop(0, RPT // 16, zdrow, 0)
      pltpu.sync_copy(zdeg, deg_sh.at[pl.ds(row0, RPT)])
      for k in range(112 // 16):
        ones_v[pl.ds(k * 16, 16)] = ones16
    stage.wait()
    ldsrc.wait()
    pltpu.sync_copy(dst_hbm.at[wid], dst_v)
    plsc.subcore_barrier()

    def g_start(j, buf, sem):
      pltpu.async_copy(p_sh.at[src_v.at[j]], buf, sem)

    def g_wait(j, buf, sem):
      pltpu.make_async_copy(p_sh.at[src_v.at[j]], buf, sem).wait()

    def consume(j, buf):
      pltpu.sync_copy(buf, acc_sh.at[dst_v.at[j]], add=True)
      if with_deg:
        pltpu.sync_copy(ones_v.at[pl.ds(0, B)], deg_sh.at[dst_v.at[j]],
                        add=True)

    g_start(0, rows0, sem0)
    g_start(1, rows1, sem1)

    def step(i, carry):
      j0 = 2 * i
      g_wait(j0, rows0, sem0)
      consume(j0, rows0)
      g_start(j0 + 2, rows0, sem0)
      g_wait(j0 + 1, rows1, sem1)
      consume(j0 + 1, rows1)
      g_start(j0 + 3, rows1, sem1)
      return carry

    lax.fori_loop(0, NCHUNK // 2 - 1, step, 0)
    g_wait(NCHUNK - 2, rows0, sem0)
    consume(NCHUNK - 2, rows0)
    g_wait(NCHUNK - 1, rows1, sem1)
    consume(NCHUNK - 1, rows1)
    plsc.subcore_barrier()

    pltpu.sync_copy(acc_sh.at[pl.ds(row0, RPT)],
                    acc_out.at[c, pl.ds(row0, RPT)])
    if with_deg:
      pltpu.sync_copy(deg_sh.at[pl.ds(row0, RPT)],
                      deg_out.at[c, pl.ds(row0, RPT)])

  return pl.kernel(body, out_type=out_type, mesh=mesh, scratch_types=scratch,
                   compiler_params=pltpu.CompilerParams(
                       use_tc_tiling_on_sc=False))


_seg_sum_l1 = _make_seg_sum(32, with_deg=True)
_seg_sum_l2 = _make_seg_sum(16, with_deg=False)


def _proj_body(x_ref, wl_ref, wr_ref, l_o, r_o):
  xv = x_ref[...]
  l_o[...] = jnp.dot(xv, wl_ref[...], preferred_element_type=_f32)
  r_o[...] = jnp.dot(xv, wr_ref[...], preferred_element_type=_f32)


def _mid_body(s1p_ref, d0_ref, d1_ref, p1r_ref, b1_ref, w2l_ref, w2r_ref,
              p2l_o, p2r_o, rdeg_o):
  deg = jnp.maximum(d0_ref[...] + d1_ref[...], 1.0)
  rdeg = 1.0 / deg
  s1 = s1p_ref[0, :N, :] + s1p_ref[1, :N, :]
  h = s1 * rdeg + b1_ref[...] + p1r_ref[...]
  h = jnp.maximum(h, 0.0)
  p2l_o[...] = jnp.dot(h, w2l_ref[...], preferred_element_type=_f32)
  p2r_o[...] = jnp.dot(h, w2r_ref[...], preferred_element_type=_f32)
  rdeg_o[...] = rdeg


def _dec_body(s2p_ref, rdeg_ref, p2r_ref, b2_ref, wd_ref, bd_ref,
              xr_o, z_o):
  s2 = s2p_ref[0, :N, :] + s2p_ref[1, :N, :]
  z = s2 * rdeg_ref[...] + b2_ref[...] + p2r_ref[...]
  z_o[...] = z
  xr_o[...] = jnp.dot(z, wd_ref[...], preferred_element_type=_f32) + bd_ref[...]


def kernel(x, edge_index, W1_l, b1_l, W1_r, W2_l, b2_l, W2_r, W_dec, b_dec):
  src = edge_index[0].astype(jnp.int32).reshape(NW, NCHUNK, B)
  dst = edge_index[1].astype(jnp.int32).reshape(NW, NCHUNK, B)

  p1l, p1r = pl.pallas_call(
      _proj_body,
      out_shape=[jax.ShapeDtypeStruct((N, 32), _f32),
                 jax.ShapeDtypeStruct((N, 32), _f32)],
  )(x, W1_l.T, W1_r.T)

  s1p, degp = _seg_sum_l1(p1l, src, dst)
  d0 = degp[0, :N, None]
  d1 = degp[1, :N, None]

  p2l, p2r, rdeg = pl.pallas_call(
      _mid_body,
      out_shape=[jax.ShapeDtypeStruct((N, 16), _f32),
                 jax.ShapeDtypeStruct((N, 16), _f32),
                 jax.ShapeDtypeStruct((N, 1), _f32)],
  )(s1p, d0, d1, p1r, b1_l[None, :], W2_l.T, W2_r.T)

  (s2p,) = _seg_sum_l2(p2l, src, dst)

  x_recon, z = pl.pallas_call(
      _dec_body,
      out_shape=[jax.ShapeDtypeStruct((N, 128), _f32),
                 jax.ShapeDtypeStruct((N, 16), _f32)],
  )(s2p, rdeg, p2r, b2_l[None, :], W_dec.T, b_dec[None, :])

  return (x_recon, z)
